# baseline (speedup 1.0000x reference)
import functools

import jax
import jax.numpy as jnp
from jax.experimental import pallas as pl

_BT = 512


def _router_body(x_ref, wt_ref, b_ref, o_ref):
    logits = jnp.dot(x_ref[...], wt_ref[...],
                     preferred_element_type=jnp.float32) + b_ref[...]
    m = jnp.max(logits, axis=-1, keepdims=True)
    e = jnp.exp(logits - m)
    o_ref[...] = e / jnp.sum(e, axis=-1, keepdims=True)


@functools.partial(jax.jit, static_argnames=())
def kernel(x, W, b):
    n_tokens, embed_dim = x.shape
    n_experts = W.shape[0]
    wt = W.T
    b2 = b.reshape(1, n_experts)
    grid = (n_tokens // _BT,)
    return pl.pallas_call(
        _router_body,
        grid=grid,
        in_specs=[
            pl.BlockSpec((_BT, embed_dim), lambda i: (i, 0)),
            pl.BlockSpec((embed_dim, n_experts), lambda i: (0, 0)),
            pl.BlockSpec((1, n_experts), lambda i: (0, 0)),
        ],
        out_specs=pl.BlockSpec((_BT, n_experts), lambda i: (i, 0)),
        out_shape=jax.ShapeDtypeStruct((n_tokens, n_experts), jnp.float32),
    )(x, wt, b2)


# speedup vs baseline: 1.1972x; 1.1972x over previous
import functools

import jax
import jax.numpy as jnp
from jax.experimental import pallas as pl

_BT = 1024


def _router_body(x_ref, wt_ref, b_ref, o_ref):
    logits = jnp.dot(x_ref[...], wt_ref[...],
                     preferred_element_type=jnp.float32) + b_ref[...]
    m = jnp.max(logits, axis=-1, keepdims=True)
    e = jnp.exp(logits - m)
    o_ref[...] = e / jnp.sum(e, axis=-1, keepdims=True)


@functools.partial(jax.jit, static_argnames=())
def kernel(x, W, b):
    n_tokens, embed_dim = x.shape
    n_experts = W.shape[0]
    wt = W.T
    b2 = b.reshape(1, n_experts)
    grid = (n_tokens // _BT,)
    return pl.pallas_call(
        _router_body,
        grid=grid,
        in_specs=[
            pl.BlockSpec((_BT, embed_dim), lambda i: (i, 0)),
            pl.BlockSpec((embed_dim, n_experts), lambda i: (0, 0)),
            pl.BlockSpec((1, n_experts), lambda i: (0, 0)),
        ],
        out_specs=pl.BlockSpec((_BT, n_experts), lambda i: (i, 0)),
        out_shape=jax.ShapeDtypeStruct((n_tokens, n_experts), jnp.float32),
    )(x, wt, b2)
